# baseline (speedup 1.0000x reference)
import functools
import jax
import jax.numpy as jnp
from jax import lax
from jax.experimental import pallas as pl
from jax.experimental.pallas import tpu as pltpu
from jax.experimental.pallas import tpu_sc as plsc

KNN = 4
RW = 8
NCAND = 2 * RW + 1
BN = 4
HW = 48
N = HW * HW
NT = BN * N
IN_CH = 256
HID = 512
OUT_CH = 256
CH = 256
NCHUNK = N // CH
BIGD = 3.0e38
BIGI = 4.0e9

NWORK = 32
RPW = NT // NWORK
WPB = N // RPW


def _rank_body(fh_ref, fv_ref, rk_ref):
    b = pl.program_id(0)
    fi = fh_ref[0]
    acc = jnp.zeros((1, N), jnp.float32)
    for c in range(NCHUNK):
        fj = fv_ref[0, pl.ds(c * CH, CH), :]
        jio = lax.broadcasted_iota(jnp.int32, (CH, N), 0) + (c * CH)
        iio = lax.broadcasted_iota(jnp.int32, (CH, N), 1)
        lt = (fj < fi).astype(jnp.float32)
        eq = jnp.logical_and(fj == fi, jio < iio).astype(jnp.float32)
        acc = acc + jnp.sum(lt + eq, axis=0, keepdims=True)
    rk_ref[0] = (acc + (b * N).astype(jnp.float32)).astype(jnp.int32)


def _build_rank(fh, fv):
    return pl.pallas_call(
        _rank_body,
        grid=(BN,),
        in_specs=[pl.BlockSpec((1, 1, N), lambda b: (b, 0, 0)),
                  pl.BlockSpec((1, N, 1), lambda b: (b, 0, 0))],
        out_specs=pl.BlockSpec((1, 1, N), lambda b: (b, 0, 0)),
        out_shape=jax.ShapeDtypeStruct((BN, 1, N), jnp.int32),
    )(fh, fv)


def _prep_body(x_hbm, rk_hbm, f_hbm, xs_out, fs_out, sg_out,
               rk_v, f_v, si_v, fsv_v, rows_v, sem):
    w = lax.axis_index("s") * 2 + lax.axis_index("c")
    gbase = w * RPW
    b = w // WPB
    pltpu.sync_copy(rk_hbm.at[pl.ds(b * N, N)], rk_v)
    pltpu.sync_copy(f_hbm.at[pl.ds(b * N, N)], f_v)

    def body(c, carry):
        rkv = rk_v[pl.ds(c * 16, 16)]
        m = jnp.logical_and(rkv >= gbase, rkv < gbase + RPW)
        tgt = jnp.where(m, rkv - gbase, 0)
        fv = f_v[pl.ds(c * 16, 16)]
        iv = lax.iota(jnp.int32, 16) + (c * 16 + b * N)
        plsc.store_scatter(fsv_v, [tgt], fv, mask=m)
        plsc.store_scatter(si_v, [tgt], iv, mask=m)
        return carry
    lax.fori_loop(0, N // 16, body, 0)

    pltpu.sync_copy(fsv_v, fs_out.at[pl.ds(gbase, RPW)])
    pltpu.sync_copy(si_v, sg_out.at[pl.ds(gbase, RPW)])
    pltpu.async_copy(x_hbm.at[si_v], rows_v, sem).wait()
    pltpu.sync_copy(rows_v, xs_out.at[pl.ds(gbase, RPW)])


@functools.cache
def _prep():
    return pl.kernel(
        _prep_body,
        out_type=(jax.ShapeDtypeStruct((NT, IN_CH), jnp.float32),
                  jax.ShapeDtypeStruct((NT,), jnp.float32),
                  jax.ShapeDtypeStruct((NT,), jnp.int32)),
        mesh=plsc.VectorSubcoreMesh(core_axis_name="c", subcore_axis_name="s"),
        scratch_types=[pltpu.VMEM((N,), jnp.int32),
                       pltpu.VMEM((N,), jnp.float32),
                       pltpu.VMEM((RPW,), jnp.int32),
                       pltpu.VMEM((RPW,), jnp.float32),
                       pltpu.VMEM((RPW, IN_CH), jnp.float32),
                       pltpu.SemaphoreType.DMA],
        compiler_params=pltpu.CompilerParams(needs_layout_passes=False))


def _gcn_body(fs_ref, sg_ref, xs_ref, w1_ref, b1_ref, w2_ref, b2_ref,
              out_ref, zpad_ref, zpad2_ref, h_ref):
    f = fs_ref[0, 0, :]
    sgf = sg_ref[0, 0, :].astype(jnp.float32)
    padd = jnp.full((RW,), BIGD, jnp.float32)
    padi = jnp.full((RW,), BIGI, jnp.float32)
    pf = jnp.concatenate([padd, f, padd])
    pidx = jnp.concatenate([padi, sgf, padi])

    ds, idxs = [], []
    for o in range(-RW, RW + 1):
        fo = lax.slice_in_dim(pf, RW + o, RW + o + N)
        ds.append(jnp.abs(fo - f))
        idxs.append(lax.slice_in_dim(pidx, RW + o, RW + o + N))

    act = [jnp.ones((N,), jnp.bool_)] * NCAND
    sel = [None] * NCAND
    for it in range(KNN + 1):
        dmin = None
        for o in range(NCAND):
            da = jnp.where(act[o], ds[o], BIGD)
            dmin = da if dmin is None else jnp.minimum(dmin, da)
        imin = None
        for o in range(NCAND):
            ia = jnp.where(jnp.logical_and(act[o], ds[o] == dmin),
                           idxs[o], BIGI)
            imin = ia if imin is None else jnp.minimum(imin, ia)
        for o in range(NCAND):
            hit = jnp.logical_and(
                act[o], jnp.logical_and(ds[o] == dmin, idxs[o] == imin))
            if it > 0:
                sel[o] = hit if sel[o] is None else jnp.logical_or(sel[o], hit)
            act[o] = jnp.logical_and(act[o], jnp.logical_not(hit))

    zz = jnp.zeros((RW,), jnp.float32)
    mf = [sel[o].astype(jnp.float32) for o in range(NCAND)]
    indeg = jnp.zeros((N,), jnp.float32)
    for o in range(-RW, RW + 1):
        pm = jnp.concatenate([zz, mf[o + RW], zz])
        indeg = indeg + lax.slice_in_dim(pm, RW - o, RW - o + N)
    deg = indeg + 2.0
    dinv = 1.0 / jnp.sqrt(deg)

    planes = []
    for o in range(-RW, RW + 1):
        c = mf[o + RW] * dinv
        if o == 0:
            c = c + 2.0 * dinv
        pc = jnp.concatenate([zz, c, zz])
        planes.append(lax.slice_in_dim(pc, RW - o, RW - o + N) * dinv)

    NB = 128
    NQ = N // NB
    WIN = NB + 2 * RW
    io_s = lax.broadcasted_iota(jnp.int32, (WIN, NB), 0)
    io_r = lax.broadcasted_iota(jnp.int32, (WIN, NB), 1)
    ohs = [(io_s - io_r == (RW - o)).astype(jnp.bfloat16)
           for o in range(-RW, RW + 1)]
    planes = [p.astype(jnp.bfloat16) for p in planes]
    bands = []
    for q in range(NQ):
        m = jnp.zeros((WIN, NB), jnp.bfloat16)
        for oi in range(NCAND):
            m = m + planes[oi][q * NB:(q + 1) * NB][None, :] * ohs[oi]
        bands.append(m)

    def dott(a, bm):
        return lax.dot_general(a, bm, (((0,), (0,)), ((), ())),
                               preferred_element_type=jnp.float32,
                               precision=lax.Precision.DEFAULT)

    def dot(a, bm):
        return lax.dot_general(a, bm, (((1,), (0,)), ((), ())),
                               preferred_element_type=jnp.float32,
                               precision=lax.Precision.DEFAULT)

    xw1 = dot(xs_ref[:], w1_ref[:])
    zpad_ref[pl.ds(0, RW), :] = jnp.zeros((RW, HID), jnp.bfloat16)
    zpad_ref[pl.ds(RW, N), :] = xw1.astype(jnp.bfloat16)
    zpad_ref[pl.ds(RW + N, RW), :] = jnp.zeros((RW, HID), jnp.bfloat16)
    zpad2_ref[pl.ds(0, RW), :] = jnp.zeros((RW, OUT_CH), jnp.bfloat16)
    zpad2_ref[pl.ds(RW + N, RW), :] = jnp.zeros((RW, OUT_CH), jnp.bfloat16)
    for q in range(NQ):
        agg = dott(bands[q], zpad_ref[pl.ds(q * NB, WIN), :])
        h_ref[pl.ds(q * NB, NB), :] = jnp.maximum(
            agg + b1_ref[:][None, :], 0.0)
    zpad2_ref[pl.ds(RW, N), :] = dot(h_ref[:], w2_ref[:]).astype(jnp.bfloat16)
    for q in range(NQ):
        agg = dott(bands[q], zpad2_ref[pl.ds(q * NB, WIN), :])
        out_ref[pl.ds(q * NB, NB), :] = jnp.maximum(
            agg + b2_ref[:][None, :], 0.0)


def _gcn(fsf, sgf, xs, w1, b1, w2, b2):
    return pl.pallas_call(
        _gcn_body,
        grid=(BN,),
        in_specs=[
            pl.BlockSpec((1, 1, N), lambda b: (b, 0, 0)),
            pl.BlockSpec((1, 1, N), lambda b: (b, 0, 0)),
            pl.BlockSpec((N, IN_CH), lambda b: (b, 0)),
            pl.BlockSpec((IN_CH, HID), lambda b: (0, 0)),
            pl.BlockSpec((HID,), lambda b: (0,)),
            pl.BlockSpec((HID, OUT_CH), lambda b: (0, 0)),
            pl.BlockSpec((OUT_CH,), lambda b: (0,)),
        ],
        out_specs=pl.BlockSpec((N, OUT_CH), lambda b: (b, 0)),
        out_shape=jax.ShapeDtypeStruct((NT, OUT_CH), jnp.float32),
        scratch_shapes=[pltpu.VMEM((N + 2 * RW, HID), jnp.bfloat16),
                        pltpu.VMEM((N + 2 * RW, OUT_CH), jnp.bfloat16),
                        pltpu.VMEM((N, HID), jnp.float32)],
    )(fsf, sgf, xs, w1, b1, w2, b2)


def _unperm_body(tab_hbm, rk_hbm, out_hbm, idx_v, rows_v, sem):
    w = lax.axis_index("s") * 2 + lax.axis_index("c")
    gbase = w * RPW
    pltpu.sync_copy(rk_hbm.at[pl.ds(gbase, RPW)], idx_v)
    pltpu.async_copy(tab_hbm.at[idx_v], rows_v, sem).wait()
    pltpu.sync_copy(rows_v, out_hbm.at[pl.ds(gbase, RPW)])


@functools.cache
def _unperm():
    return pl.kernel(
        _unperm_body,
        out_type=jax.ShapeDtypeStruct((NT, OUT_CH), jnp.float32),
        mesh=plsc.VectorSubcoreMesh(core_axis_name="c", subcore_axis_name="s"),
        scratch_types=[pltpu.VMEM((RPW,), jnp.int32),
                       pltpu.VMEM((RPW, OUT_CH), jnp.float32),
                       pltpu.SemaphoreType.DMA],
        compiler_params=pltpu.CompilerParams(needs_layout_passes=False))


def kernel(density_maps, feature_maps, W1, b1, W2, b2):
    f = density_maps.reshape(BN, N)
    rk3 = _build_rank(f.reshape(BN, 1, N), f.reshape(BN, N, 1))
    rkf = rk3.reshape(NT)
    x = feature_maps.transpose(0, 2, 3, 1).reshape(NT, IN_CH)
    xs, fsf, sgf = _prep()(x, rkf, f.reshape(NT))
    out_s = _gcn(fsf.reshape(BN, 1, N), sgf.reshape(BN, 1, N), xs,
                 W1, b1, W2, b2)
    out = _unperm()(out_s, rkf)
    return out.reshape(BN, HW, HW, OUT_CH).transpose(0, 3, 1, 2)


# speedup vs baseline: 1.0111x; 1.0111x over previous
import functools
import jax
import jax.numpy as jnp
from jax import lax
from jax.experimental import pallas as pl
from jax.experimental.pallas import tpu as pltpu
from jax.experimental.pallas import tpu_sc as plsc

KNN = 4
RW = 8
NCAND = 2 * RW + 1
BN = 4
HW = 48
N = HW * HW
NT = BN * N
IN_CH = 256
HID = 512
OUT_CH = 256
CH = 256
NCHUNK = N // CH
BIGD = 3.0e38
BIGI = 4.0e9

NWORK = 32
RPW = NT // NWORK
WPB = N // RPW


def _rank_body(fh_ref, fv_ref, rk_ref):
    b = pl.program_id(0)
    fi = fh_ref[0]
    acc = jnp.zeros((1, N), jnp.float32)
    for c in range(NCHUNK):
        fj = fv_ref[0, pl.ds(c * CH, CH), :]
        jio = lax.broadcasted_iota(jnp.int32, (CH, N), 0) + (c * CH)
        iio = lax.broadcasted_iota(jnp.int32, (CH, N), 1)
        lt = (fj < fi).astype(jnp.float32)
        eq = jnp.logical_and(fj == fi, jio < iio).astype(jnp.float32)
        acc = acc + jnp.sum(lt + eq, axis=0, keepdims=True)
    rk_ref[0] = (acc + (b * N).astype(jnp.float32)).astype(jnp.int32)


def _build_rank(fh, fv):
    return pl.pallas_call(
        _rank_body,
        grid=(BN,),
        in_specs=[pl.BlockSpec((1, 1, N), lambda b: (b, 0, 0)),
                  pl.BlockSpec((1, N, 1), lambda b: (b, 0, 0))],
        out_specs=pl.BlockSpec((1, 1, N), lambda b: (b, 0, 0)),
        out_shape=jax.ShapeDtypeStruct((BN, 1, N), jnp.int32),
    )(fh, fv)


def _prep_body(x_hbm, rk_hbm, f_hbm, xs_out, fs_out, sg_out,
               rk_v, f_v, si_v, fsv_v, rows_v, sem):
    w = lax.axis_index("s") * 2 + lax.axis_index("c")
    gbase = w * RPW
    b = w // WPB
    pltpu.sync_copy(rk_hbm.at[pl.ds(b * N, N)], rk_v)
    pltpu.sync_copy(f_hbm.at[pl.ds(b * N, N)], f_v)

    def body(c, carry):
        rkv = rk_v[pl.ds(c * 16, 16)]
        m = jnp.logical_and(rkv >= gbase, rkv < gbase + RPW)
        tgt = jnp.where(m, rkv - gbase, 0)
        fv = f_v[pl.ds(c * 16, 16)]
        iv = lax.iota(jnp.int32, 16) + (c * 16 + b * N)
        plsc.store_scatter(fsv_v, [tgt], fv, mask=m)
        plsc.store_scatter(si_v, [tgt], iv, mask=m)
        return carry
    lax.fori_loop(0, N // 16, body, 0)

    pltpu.sync_copy(fsv_v, fs_out.at[pl.ds(gbase, RPW)])
    pltpu.sync_copy(si_v, sg_out.at[pl.ds(gbase, RPW)])
    pltpu.async_copy(x_hbm.at[si_v], rows_v, sem).wait()
    pltpu.sync_copy(rows_v, xs_out.at[pl.ds(gbase, RPW)])


@functools.cache
def _prep():
    return pl.kernel(
        _prep_body,
        out_type=(jax.ShapeDtypeStruct((NT, IN_CH), jnp.float32),
                  jax.ShapeDtypeStruct((NT,), jnp.float32),
                  jax.ShapeDtypeStruct((NT,), jnp.int32)),
        mesh=plsc.VectorSubcoreMesh(core_axis_name="c", subcore_axis_name="s"),
        scratch_types=[pltpu.VMEM((N,), jnp.int32),
                       pltpu.VMEM((N,), jnp.float32),
                       pltpu.VMEM((RPW,), jnp.int32),
                       pltpu.VMEM((RPW,), jnp.float32),
                       pltpu.VMEM((RPW, IN_CH), jnp.float32),
                       pltpu.SemaphoreType.DMA],
        compiler_params=pltpu.CompilerParams(needs_layout_passes=False))


def _gcn_body(fs_ref, sg_ref, xs_ref, w1_ref, b1_ref, w2_ref, b2_ref,
              out_ref, zpad_ref, zpad2_ref):
    f = fs_ref[0, 0, :]
    sgf = sg_ref[0, 0, :].astype(jnp.float32)
    padd = jnp.full((RW,), BIGD, jnp.float32)
    padi = jnp.full((RW,), BIGI, jnp.float32)
    pf = jnp.concatenate([padd, f, padd])
    pidx = jnp.concatenate([padi, sgf, padi])

    ds, idxs = [], []
    for o in range(-RW, RW + 1):
        fo = lax.slice_in_dim(pf, RW + o, RW + o + N)
        ds.append(jnp.abs(fo - f))
        idxs.append(lax.slice_in_dim(pidx, RW + o, RW + o + N))

    act = [jnp.ones((N,), jnp.bool_)] * NCAND
    sel = [None] * NCAND
    for it in range(KNN + 1):
        dmin = None
        for o in range(NCAND):
            da = jnp.where(act[o], ds[o], BIGD)
            dmin = da if dmin is None else jnp.minimum(dmin, da)
        imin = None
        for o in range(NCAND):
            ia = jnp.where(jnp.logical_and(act[o], ds[o] == dmin),
                           idxs[o], BIGI)
            imin = ia if imin is None else jnp.minimum(imin, ia)
        for o in range(NCAND):
            hit = jnp.logical_and(
                act[o], jnp.logical_and(ds[o] == dmin, idxs[o] == imin))
            if it > 0:
                sel[o] = hit if sel[o] is None else jnp.logical_or(sel[o], hit)
            act[o] = jnp.logical_and(act[o], jnp.logical_not(hit))

    zz = jnp.zeros((RW,), jnp.float32)
    mf = [sel[o].astype(jnp.float32) for o in range(NCAND)]
    indeg = jnp.zeros((N,), jnp.float32)
    for o in range(-RW, RW + 1):
        pm = jnp.concatenate([zz, mf[o + RW], zz])
        indeg = indeg + lax.slice_in_dim(pm, RW - o, RW - o + N)
    deg = indeg + 2.0
    dinv = 1.0 / jnp.sqrt(deg)

    planes = []
    for o in range(-RW, RW + 1):
        c = mf[o + RW] * dinv
        if o == 0:
            c = c + 2.0 * dinv
        pc = jnp.concatenate([zz, c, zz])
        planes.append(lax.slice_in_dim(pc, RW - o, RW - o + N) * dinv)

    NB = 128
    NQ = N // NB
    WIN = NB + 2 * RW
    io_s = lax.broadcasted_iota(jnp.int32, (WIN, NB), 0)
    io_r = lax.broadcasted_iota(jnp.int32, (WIN, NB), 1)
    ohs = [(io_s - io_r == (RW - o)).astype(jnp.bfloat16)
           for o in range(-RW, RW + 1)]
    planes = [p.astype(jnp.bfloat16) for p in planes]
    bands = []
    for q in range(NQ):
        m = jnp.zeros((WIN, NB), jnp.bfloat16)
        for oi in range(NCAND):
            m = m + planes[oi][q * NB:(q + 1) * NB][None, :] * ohs[oi]
        bands.append(m)

    def dott(a, bm):
        return lax.dot_general(a, bm, (((0,), (0,)), ((), ())),
                               preferred_element_type=jnp.float32,
                               precision=lax.Precision.DEFAULT)

    def dot(a, bm):
        return lax.dot_general(a, bm, (((1,), (0,)), ((), ())),
                               preferred_element_type=jnp.float32,
                               precision=lax.Precision.DEFAULT)

    xw1 = dot(xs_ref[:], w1_ref[:])
    zpad_ref[pl.ds(0, RW), :] = jnp.zeros((RW, HID), jnp.bfloat16)
    zpad_ref[pl.ds(RW, N), :] = xw1.astype(jnp.bfloat16)
    zpad_ref[pl.ds(RW + N, RW), :] = jnp.zeros((RW, HID), jnp.bfloat16)
    zpad2_ref[pl.ds(0, RW), :] = jnp.zeros((RW, OUT_CH), jnp.bfloat16)
    zpad2_ref[pl.ds(RW + N, RW), :] = jnp.zeros((RW, OUT_CH), jnp.bfloat16)
    for q in range(NQ):
        agg = dott(bands[q], zpad_ref[pl.ds(q * NB, WIN), :])
        hq = jnp.maximum(agg + b1_ref[:][None, :], 0.0)
        zpad2_ref[pl.ds(RW + q * NB, NB), :] = dot(
            hq, w2_ref[:]).astype(jnp.bfloat16)
    for q in range(NQ):
        agg = dott(bands[q], zpad2_ref[pl.ds(q * NB, WIN), :])
        out_ref[pl.ds(q * NB, NB), :] = jnp.maximum(
            agg + b2_ref[:][None, :], 0.0)


def _gcn(fsf, sgf, xs, w1, b1, w2, b2):
    return pl.pallas_call(
        _gcn_body,
        grid=(BN,),
        in_specs=[
            pl.BlockSpec((1, 1, N), lambda b: (b, 0, 0)),
            pl.BlockSpec((1, 1, N), lambda b: (b, 0, 0)),
            pl.BlockSpec((N, IN_CH), lambda b: (b, 0)),
            pl.BlockSpec((IN_CH, HID), lambda b: (0, 0)),
            pl.BlockSpec((HID,), lambda b: (0,)),
            pl.BlockSpec((HID, OUT_CH), lambda b: (0, 0)),
            pl.BlockSpec((OUT_CH,), lambda b: (0,)),
        ],
        out_specs=pl.BlockSpec((N, OUT_CH), lambda b: (b, 0)),
        out_shape=jax.ShapeDtypeStruct((NT, OUT_CH), jnp.float32),
        scratch_shapes=[pltpu.VMEM((N + 2 * RW, HID), jnp.bfloat16),
                        pltpu.VMEM((N + 2 * RW, OUT_CH), jnp.bfloat16)],
    )(fsf, sgf, xs, w1, b1, w2, b2)


def _unperm_body(tab_hbm, rk_hbm, out_hbm, idx_v, rows_v, sem):
    w = lax.axis_index("s") * 2 + lax.axis_index("c")
    gbase = w * RPW
    pltpu.sync_copy(rk_hbm.at[pl.ds(gbase, RPW)], idx_v)
    pltpu.async_copy(tab_hbm.at[idx_v], rows_v, sem).wait()
    pltpu.sync_copy(rows_v, out_hbm.at[pl.ds(gbase, RPW)])


@functools.cache
def _unperm():
    return pl.kernel(
        _unperm_body,
        out_type=jax.ShapeDtypeStruct((NT, OUT_CH), jnp.float32),
        mesh=plsc.VectorSubcoreMesh(core_axis_name="c", subcore_axis_name="s"),
        scratch_types=[pltpu.VMEM((RPW,), jnp.int32),
                       pltpu.VMEM((RPW, OUT_CH), jnp.float32),
                       pltpu.SemaphoreType.DMA],
        compiler_params=pltpu.CompilerParams(needs_layout_passes=False))


def kernel(density_maps, feature_maps, W1, b1, W2, b2):
    f = density_maps.reshape(BN, N)
    rk3 = _build_rank(f.reshape(BN, 1, N), f.reshape(BN, N, 1))
    rkf = rk3.reshape(NT)
    x = feature_maps.transpose(0, 2, 3, 1).reshape(NT, IN_CH)
    xs, fsf, sgf = _prep()(x, rkf, f.reshape(NT))
    out_s = _gcn(fsf.reshape(BN, 1, N), sgf.reshape(BN, 1, N), xs,
                 W1, b1, W2, b2)
    out = _unperm()(out_s, rkf)
    return out.reshape(BN, HW, HW, OUT_CH).transpose(0, 3, 1, 2)


# speedup vs baseline: 1.0874x; 1.0755x over previous
import functools
import jax
import jax.numpy as jnp
from jax import lax
from jax.experimental import pallas as pl
from jax.experimental.pallas import tpu as pltpu
from jax.experimental.pallas import tpu_sc as plsc

KNN = 4
RW = 8
NCAND = 2 * RW + 1
BN = 4
HW = 48
N = HW * HW
NT = BN * N
IN_CH = 256
HID = 512
OUT_CH = 256
CH = 256
NCHUNK = N // CH
BIGD = 3.0e38
BIGI = 4.0e9

NWORK = 32
RPW = NT // NWORK
WPB = N // RPW


def _rank_body(fh_ref, fv_ref, rk_ref):
    accs = [jnp.zeros((1, N), jnp.float32) for _ in range(BN)]
    for c in range(NCHUNK):
        jio = lax.broadcasted_iota(jnp.int32, (CH, N), 0) + (c * CH)
        iio = lax.broadcasted_iota(jnp.int32, (CH, N), 1)
        jlt = jio < iio
        for b in range(BN):
            fj = fv_ref[b, pl.ds(c * CH, CH), :]
            fi = fh_ref[b]
            cmb = jnp.logical_or(fj < fi,
                                 jnp.logical_and(fj == fi, jlt))
            accs[b] = accs[b] + jnp.sum(cmb.astype(jnp.float32),
                                        axis=0, keepdims=True)
    for b in range(BN):
        rk_ref[b] = (accs[b] + float(b * N)).astype(jnp.int32)


def _build_rank(fh, fv):
    return pl.pallas_call(
        _rank_body,
        out_shape=jax.ShapeDtypeStruct((BN, 1, N), jnp.int32),
    )(fh, fv)


def _prep_body(x_hbm, rk_hbm, f_hbm, xs_out, fs_out, sg_out,
               rk_v, f_v, si_v, fsv_v, rows_v, sem):
    w = lax.axis_index("s") * 2 + lax.axis_index("c")
    gbase = w * RPW
    b = w // WPB
    pltpu.sync_copy(rk_hbm.at[pl.ds(b * N, N)], rk_v)
    pltpu.sync_copy(f_hbm.at[pl.ds(b * N, N)], f_v)

    def body(c, carry):
        rkv = rk_v[pl.ds(c * 16, 16)]
        m = jnp.logical_and(rkv >= gbase, rkv < gbase + RPW)
        tgt = jnp.where(m, rkv - gbase, 0)
        fv = f_v[pl.ds(c * 16, 16)]
        iv = lax.iota(jnp.int32, 16) + (c * 16 + b * N)
        plsc.store_scatter(fsv_v, [tgt], fv, mask=m)
        plsc.store_scatter(si_v, [tgt], iv, mask=m)
        return carry
    lax.fori_loop(0, N // 16, body, 0)

    pltpu.sync_copy(fsv_v, fs_out.at[pl.ds(gbase, RPW)])
    pltpu.sync_copy(si_v, sg_out.at[pl.ds(gbase, RPW)])
    pltpu.async_copy(x_hbm.at[si_v], rows_v, sem).wait()
    pltpu.sync_copy(rows_v, xs_out.at[pl.ds(gbase, RPW)])


@functools.cache
def _prep():
    return pl.kernel(
        _prep_body,
        out_type=(jax.ShapeDtypeStruct((NT, IN_CH), jnp.float32),
                  jax.ShapeDtypeStruct((NT,), jnp.float32),
                  jax.ShapeDtypeStruct((NT,), jnp.int32)),
        mesh=plsc.VectorSubcoreMesh(core_axis_name="c", subcore_axis_name="s"),
        scratch_types=[pltpu.VMEM((N,), jnp.int32),
                       pltpu.VMEM((N,), jnp.float32),
                       pltpu.VMEM((RPW,), jnp.int32),
                       pltpu.VMEM((RPW,), jnp.float32),
                       pltpu.VMEM((RPW, IN_CH), jnp.float32),
                       pltpu.SemaphoreType.DMA],
        compiler_params=pltpu.CompilerParams(needs_layout_passes=False))


def _gcn_body(fs_ref, sg_ref, xs_ref, w1_ref, b1_ref, w2_ref, b2_ref,
              out_ref, zpad_ref, zpad2_ref):
    f = fs_ref[0, 0, :]
    sgf = sg_ref[0, 0, :].astype(jnp.float32)
    padd = jnp.full((RW,), BIGD, jnp.float32)
    padi = jnp.full((RW,), BIGI, jnp.float32)
    pf = jnp.concatenate([padd, f, padd])
    pidx = jnp.concatenate([padi, sgf, padi])

    ds, idxs = [], []
    for o in range(-RW, RW + 1):
        fo = lax.slice_in_dim(pf, RW + o, RW + o + N)
        ds.append(jnp.abs(fo - f))
        idxs.append(lax.slice_in_dim(pidx, RW + o, RW + o + N))

    act = [jnp.ones((N,), jnp.bool_)] * NCAND
    sel = [None] * NCAND
    for it in range(KNN + 1):
        dmin = None
        for o in range(NCAND):
            da = jnp.where(act[o], ds[o], BIGD)
            dmin = da if dmin is None else jnp.minimum(dmin, da)
        imin = None
        for o in range(NCAND):
            ia = jnp.where(jnp.logical_and(act[o], ds[o] == dmin),
                           idxs[o], BIGI)
            imin = ia if imin is None else jnp.minimum(imin, ia)
        for o in range(NCAND):
            hit = jnp.logical_and(
                act[o], jnp.logical_and(ds[o] == dmin, idxs[o] == imin))
            if it > 0:
                sel[o] = hit if sel[o] is None else jnp.logical_or(sel[o], hit)
            act[o] = jnp.logical_and(act[o], jnp.logical_not(hit))

    zz = jnp.zeros((RW,), jnp.float32)
    mf = [sel[o].astype(jnp.float32) for o in range(NCAND)]
    indeg = jnp.zeros((N,), jnp.float32)
    for o in range(-RW, RW + 1):
        pm = jnp.concatenate([zz, mf[o + RW], zz])
        indeg = indeg + lax.slice_in_dim(pm, RW - o, RW - o + N)
    deg = indeg + 2.0
    dinv = 1.0 / jnp.sqrt(deg)

    planes = []
    for o in range(-RW, RW + 1):
        c = mf[o + RW] * dinv
        if o == 0:
            c = c + 2.0 * dinv
        pc = jnp.concatenate([zz, c, zz])
        planes.append(lax.slice_in_dim(pc, RW - o, RW - o + N) * dinv)

    NB = 128
    NQ = N // NB
    WIN = NB + 2 * RW
    io_s = lax.broadcasted_iota(jnp.int32, (WIN, NB), 0)
    io_r = lax.broadcasted_iota(jnp.int32, (WIN, NB), 1)
    ohs = [(io_s - io_r == (RW - o)).astype(jnp.float32)
           for o in range(-RW, RW + 1)]
    bands = []
    for q in range(NQ):
        m = jnp.zeros((WIN, NB), jnp.float32)
        for oi in range(NCAND):
            m = m + planes[oi][q * NB:(q + 1) * NB][None, :] * ohs[oi]
        bands.append(m.astype(jnp.bfloat16))

    def dott(a, bm):
        return lax.dot_general(a, bm, (((0,), (0,)), ((), ())),
                               preferred_element_type=jnp.float32,
                               precision=lax.Precision.DEFAULT)

    def dot(a, bm):
        return lax.dot_general(a, bm, (((1,), (0,)), ((), ())),
                               preferred_element_type=jnp.float32,
                               precision=lax.Precision.DEFAULT)

    xw1 = dot(xs_ref[:], w1_ref[:])
    zpad_ref[pl.ds(0, RW), :] = jnp.zeros((RW, HID), jnp.bfloat16)
    zpad_ref[pl.ds(RW, N), :] = xw1.astype(jnp.bfloat16)
    zpad_ref[pl.ds(RW + N, RW), :] = jnp.zeros((RW, HID), jnp.bfloat16)
    zpad2_ref[pl.ds(0, RW), :] = jnp.zeros((RW, OUT_CH), jnp.bfloat16)
    zpad2_ref[pl.ds(RW + N, RW), :] = jnp.zeros((RW, OUT_CH), jnp.bfloat16)
    for q in range(NQ):
        agg = dott(bands[q], zpad_ref[pl.ds(q * NB, WIN), :])
        hq = jnp.maximum(agg + b1_ref[:][None, :], 0.0)
        zpad2_ref[pl.ds(RW + q * NB, NB), :] = dot(
            hq, w2_ref[:]).astype(jnp.bfloat16)
    for q in range(NQ):
        agg = dott(bands[q], zpad2_ref[pl.ds(q * NB, WIN), :])
        out_ref[pl.ds(q * NB, NB), :] = jnp.maximum(
            agg + b2_ref[:][None, :], 0.0)


def _gcn(fsf, sgf, xs, w1, b1, w2, b2):
    return pl.pallas_call(
        _gcn_body,
        grid=(BN,),
        in_specs=[
            pl.BlockSpec((1, 1, N), lambda b: (b, 0, 0)),
            pl.BlockSpec((1, 1, N), lambda b: (b, 0, 0)),
            pl.BlockSpec((N, IN_CH), lambda b: (b, 0)),
            pl.BlockSpec((IN_CH, HID), lambda b: (0, 0)),
            pl.BlockSpec((HID,), lambda b: (0,)),
            pl.BlockSpec((HID, OUT_CH), lambda b: (0, 0)),
            pl.BlockSpec((OUT_CH,), lambda b: (0,)),
        ],
        out_specs=pl.BlockSpec((N, OUT_CH), lambda b: (b, 0)),
        out_shape=jax.ShapeDtypeStruct((NT, OUT_CH), jnp.float32),
        scratch_shapes=[pltpu.VMEM((N + 2 * RW, HID), jnp.bfloat16),
                        pltpu.VMEM((N + 2 * RW, OUT_CH), jnp.bfloat16)],
    )(fsf, sgf, xs, w1, b1, w2, b2)


def _unperm_body(tab_hbm, rk_hbm, out_hbm, idx_v, rows_v, sem):
    w = lax.axis_index("s") * 2 + lax.axis_index("c")
    gbase = w * RPW
    pltpu.sync_copy(rk_hbm.at[pl.ds(gbase, RPW)], idx_v)
    pltpu.async_copy(tab_hbm.at[idx_v], rows_v, sem).wait()
    pltpu.sync_copy(rows_v, out_hbm.at[pl.ds(gbase, RPW)])


@functools.cache
def _unperm():
    return pl.kernel(
        _unperm_body,
        out_type=jax.ShapeDtypeStruct((NT, OUT_CH), jnp.float32),
        mesh=plsc.VectorSubcoreMesh(core_axis_name="c", subcore_axis_name="s"),
        scratch_types=[pltpu.VMEM((RPW,), jnp.int32),
                       pltpu.VMEM((RPW, OUT_CH), jnp.float32),
                       pltpu.SemaphoreType.DMA],
        compiler_params=pltpu.CompilerParams(needs_layout_passes=False))


def kernel(density_maps, feature_maps, W1, b1, W2, b2):
    f = density_maps.reshape(BN, N)
    rk3 = _build_rank(f.reshape(BN, 1, N), f.reshape(BN, N, 1))
    rkf = rk3.reshape(NT)
    x = feature_maps.transpose(0, 2, 3, 1).reshape(NT, IN_CH)
    xs, fsf, sgf = _prep()(x, rkf, f.reshape(NT))
    out_s = _gcn(fsf.reshape(BN, 1, N), sgf.reshape(BN, 1, N), xs,
                 W1, b1, W2, b2)
    out = _unperm()(out_s, rkf)
    return out.reshape(BN, HW, HW, OUT_CH).transpose(0, 3, 1, 2)
